# baseline (speedup 1.0000x reference)
import functools

import jax
import jax.numpy as jnp
from jax import lax
from jax.experimental import pallas as pl
from jax.experimental.pallas import tpu as pltpu
from jax.experimental.pallas import tpu_sc as plsc

B = 16384
N_ANIMAL, D_ANIMAL = 80, 5
N_ITEM, D_ITEM = 20, 3
NC, NS, L = 2, 16, 16
NW = NC * NS
BPW = B // NW
CHUNKS = BPW // L


@functools.cache
def _build_sc_unit():
  mesh = plsc.VectorSubcoreMesh(
      core_axis_name="c", subcore_axis_name="s", num_cores=NC, num_subcores=NS
  )

  @functools.partial(
      pl.kernel,
      out_type=(
          jax.ShapeDtypeStruct((D_ANIMAL, B), jnp.float32),
          jax.ShapeDtypeStruct((D_ITEM, B), jnp.float32),
          jax.ShapeDtypeStruct((2, B), jnp.float32),
      ),
      mesh=mesh,
      scratch_types=[
          pltpu.VMEM((BPW,), jnp.int32),
          pltpu.VMEM((BPW,), jnp.int32),
          pltpu.VMEM((BPW,), jnp.float32),
          pltpu.VMEM((BPW,), jnp.float32),
          pltpu.VMEM((N_ANIMAL * D_ANIMAL,), jnp.float32),
          pltpu.VMEM((N_ITEM * D_ITEM,), jnp.float32),
          pltpu.VMEM((6 * L,), jnp.float32),
          pltpu.VMEM((D_ANIMAL * BPW,), jnp.float32),
          pltpu.VMEM((D_ITEM * BPW,), jnp.float32),
          pltpu.VMEM((2 * BPW,), jnp.float32),
          pltpu.SemaphoreType.DMA,
      ],
      compiler_params=pltpu.CompilerParams(
          needs_layout_passes=False,
          use_tc_tiling_on_sc=False,
          disable_bounds_checks=True,
          disable_semaphore_checks=True,
      ),
  )
  def sc_unit(aid_hbm, iid_hbm, hp_hbm, ta_hbm, ti_hbm, wb_hbm,
              oa_hbm, oi_hbm, os_hbm,
              aid_v, iid_v, hp_v, atk_v, ta_v, ti_v, wb_v, oa_v, oi_v, os_v,
              sem):
    wid = lax.axis_index("s") * NC + lax.axis_index("c")
    base = wid * BPW
    cps = [
        pltpu.async_copy(aid_hbm.at[pl.ds(base, BPW)], aid_v, sem),
        pltpu.async_copy(iid_hbm.at[pl.ds(base, BPW)], iid_v, sem),
        pltpu.async_copy(hp_hbm.at[pl.ds(base, BPW)], hp_v, sem),
        pltpu.async_copy(hp_hbm.at[pl.ds(B + base, BPW)], atk_v, sem),
        pltpu.async_copy(ta_hbm, ta_v, sem),
        pltpu.async_copy(ti_hbm, ti_v, sem),
        pltpu.async_copy(wb_hbm, wb_v, sem),
    ]
    for cp in cps:
      cp.wait()

    w00 = wb_v[pl.ds(0 * L, L)]
    w01 = wb_v[pl.ds(1 * L, L)]
    w10 = wb_v[pl.ds(2 * L, L)]
    w11 = wb_v[pl.ds(3 * L, L)]
    b0 = wb_v[pl.ds(4 * L, L)]
    b1 = wb_v[pl.ds(5 * L, L)]

    @plsc.parallel_loop(0, CHUNKS, unroll=8)
    def body(c):
      off = c * L
      aid = aid_v[pl.ds(off, L)]
      iid = iid_v[pl.ds(off, L)]
      for d in range(D_ANIMAL):
        oa_v[pl.ds(d * BPW + off, L)] = plsc.load_gather(
            ta_v, [aid * D_ANIMAL + d])
      for d in range(D_ITEM):
        oi_v[pl.ds(d * BPW + off, L)] = plsc.load_gather(
            ti_v, [iid * D_ITEM + d])
      hp = hp_v[pl.ds(off, L)]
      atk = atk_v[pl.ds(off, L)]
      os_v[pl.ds(off, L)] = hp * w00 + atk * w01 + b0
      os_v[pl.ds(BPW + off, L)] = hp * w10 + atk * w11 + b1

    ocps = []
    for d in range(D_ANIMAL):
      ocps.append(pltpu.async_copy(
          oa_v.at[pl.ds(d * BPW, BPW)], oa_hbm.at[d, pl.ds(base, BPW)], sem))
    for d in range(D_ITEM):
      ocps.append(pltpu.async_copy(
          oi_v.at[pl.ds(d * BPW, BPW)], oi_hbm.at[d, pl.ds(base, BPW)], sem))
    for d in range(2):
      ocps.append(pltpu.async_copy(
          os_v.at[pl.ds(d * BPW, BPW)], os_hbm.at[d, pl.ds(base, BPW)], sem))
    for cp in ocps:
      cp.wait()

  return sc_unit


def kernel(animal_id, item_id, hp_atk, table_animal, table_item, W, b):
  wb = jnp.broadcast_to(
      jnp.concatenate([W.reshape(-1), b]).reshape(6, 1), (6, L)
  ).reshape(-1)
  oa_t, oi_t, os_t = _build_sc_unit()(
      animal_id,
      item_id,
      hp_atk.T.reshape(-1),
      table_animal.reshape(-1),
      table_item.reshape(-1),
      wb,
  )
  return (oa_t.T, oi_t.T, os_t.T)


# speedup vs baseline: 1.0308x; 1.0308x over previous
import functools

import jax
import jax.numpy as jnp
from jax import lax
from jax.experimental import pallas as pl
from jax.experimental.pallas import tpu as pltpu
from jax.experimental.pallas import tpu_sc as plsc

B = 16384
N_ANIMAL, D_ANIMAL = 80, 5
N_ITEM, D_ITEM = 20, 3
NC, NS, L = 2, 16, 16
NW = NC * NS
BPW = B // NW
CHUNKS = BPW // L


@functools.cache
def _build_sc_unit():
  mesh = plsc.VectorSubcoreMesh(
      core_axis_name="c", subcore_axis_name="s", num_cores=NC, num_subcores=NS
  )

  @functools.partial(
      pl.kernel,
      out_type=(
          jax.ShapeDtypeStruct((D_ANIMAL, B), jnp.float32),
          jax.ShapeDtypeStruct((D_ITEM, B), jnp.float32),
          jax.ShapeDtypeStruct((2, B), jnp.float32),
      ),
      mesh=mesh,
      scratch_types=[
          pltpu.VMEM((BPW,), jnp.int32),
          pltpu.VMEM((BPW,), jnp.int32),
          pltpu.VMEM((2 * BPW,), jnp.float32),
          pltpu.VMEM((560,), jnp.float32),
          pltpu.VMEM((D_ANIMAL * BPW,), jnp.float32),
          pltpu.VMEM((D_ITEM * BPW,), jnp.float32),
          pltpu.VMEM((2 * BPW,), jnp.float32),
          pltpu.SemaphoreType.DMA,
      ],
      compiler_params=pltpu.CompilerParams(
          needs_layout_passes=False,
          use_tc_tiling_on_sc=False,
          disable_bounds_checks=True,
          disable_semaphore_checks=True,
      ),
  )
  def sc_unit(aid_hbm, iid_hbm, hp_hbm, pr_hbm,
              oa_hbm, oi_hbm, os_hbm,
              aid_v, iid_v, hp_v, pr_v, oa_v, oi_v, os_v,
              sem):
    wid = lax.axis_index("s") * NC + lax.axis_index("c")
    base = wid * BPW
    cps = [
        pltpu.async_copy(aid_hbm.at[pl.ds(base, BPW)], aid_v, sem),
        pltpu.async_copy(iid_hbm.at[pl.ds(base, BPW)], iid_v, sem),
        pltpu.async_copy(hp_hbm.at[pl.ds(2 * base, 2 * BPW)], hp_v, sem),
        pltpu.async_copy(pr_hbm, pr_v, sem),
    ]
    for cp in cps:
      cp.wait()

    WB = 464
    w00 = pr_v[pl.ds(WB + 0 * L, L)]
    w01 = pr_v[pl.ds(WB + 1 * L, L)]
    w10 = pr_v[pl.ds(WB + 2 * L, L)]
    w11 = pr_v[pl.ds(WB + 3 * L, L)]
    b0 = pr_v[pl.ds(WB + 4 * L, L)]
    b1 = pr_v[pl.ds(WB + 5 * L, L)]

    @plsc.parallel_loop(0, CHUNKS, unroll=8)
    def body(c):
      off = c * L
      aid = aid_v[pl.ds(off, L)]
      iid = iid_v[pl.ds(off, L)]
      for d in range(D_ANIMAL):
        oa_v[pl.ds(d * BPW + off, L)] = plsc.load_gather(
            pr_v, [aid * D_ANIMAL + d])
      for d in range(D_ITEM):
        oi_v[pl.ds(d * BPW + off, L)] = plsc.load_gather(
            pr_v, [400 + iid * D_ITEM + d])
      gl = off // 128
      lo = off % 128
      hp = hp_v[pl.ds(gl * 256 + lo, L)]
      atk = hp_v[pl.ds(gl * 256 + 128 + lo, L)]
      os_v[pl.ds(off, L)] = hp * w00 + atk * w01 + b0
      os_v[pl.ds(BPW + off, L)] = hp * w10 + atk * w11 + b1

    ocps = []
    for d in range(D_ANIMAL):
      ocps.append(pltpu.async_copy(
          oa_v.at[pl.ds(d * BPW, BPW)], oa_hbm.at[d, pl.ds(base, BPW)], sem))
    for d in range(D_ITEM):
      ocps.append(pltpu.async_copy(
          oi_v.at[pl.ds(d * BPW, BPW)], oi_hbm.at[d, pl.ds(base, BPW)], sem))
    for d in range(2):
      ocps.append(pltpu.async_copy(
          os_v.at[pl.ds(d * BPW, BPW)], os_hbm.at[d, pl.ds(base, BPW)], sem))
    for cp in ocps:
      cp.wait()

  return sc_unit


def kernel(animal_id, item_id, hp_atk, table_animal, table_item, W, b):
  wb = jnp.broadcast_to(
      jnp.concatenate([W.reshape(-1), b]).reshape(6, 1), (6, L)
  ).reshape(-1)
  params = jnp.concatenate([
      table_animal.reshape(-1),
      table_item.reshape(-1),
      jnp.zeros((4,), jnp.float32),
      wb,
  ])
  oa_t, oi_t, os_t = _build_sc_unit()(
      animal_id,
      item_id,
      hp_atk.T.reshape(2, 128, 128).transpose(1, 0, 2).reshape(-1),
      params,
  )
  return (oa_t.T, oi_t.T, os_t.T)


# speedup vs baseline: 1.0619x; 1.0302x over previous
import functools

import jax
import jax.numpy as jnp
from jax import lax
from jax.experimental import pallas as pl
from jax.experimental.pallas import tpu as pltpu
from jax.experimental.pallas import tpu_sc as plsc

B = 16384
N_ANIMAL, D_ANIMAL = 80, 5
N_ITEM, D_ITEM = 20, 3
NC, NS, L = 2, 16, 16
NW = NC * NS
BPW = B // NW
CHUNKS = BPW // L


@functools.cache
def _build_sc_unit():
  mesh = plsc.VectorSubcoreMesh(
      core_axis_name="c", subcore_axis_name="s", num_cores=NC, num_subcores=NS
  )

  @functools.partial(
      pl.kernel,
      out_type=(
          jax.ShapeDtypeStruct((8 * B,), jnp.float32),
          jax.ShapeDtypeStruct((4 * B,), jnp.float32),
          jax.ShapeDtypeStruct((2 * B,), jnp.float32),
      ),
      mesh=mesh,
      scratch_types=[
          pltpu.VMEM((BPW,), jnp.int32),
          pltpu.VMEM((BPW,), jnp.int32),
          pltpu.VMEM((2 * BPW,), jnp.float32),
          pltpu.VMEM((560,), jnp.float32),
          pltpu.VMEM((8 * BPW,), jnp.float32),
          pltpu.VMEM((4 * BPW,), jnp.float32),
          pltpu.VMEM((2 * BPW,), jnp.float32),
          pltpu.SemaphoreType.DMA,
      ],
      compiler_params=pltpu.CompilerParams(
          needs_layout_passes=False,
          use_tc_tiling_on_sc=False,
          disable_bounds_checks=True,
          disable_semaphore_checks=True,
      ),
  )
  def sc_unit(aid_hbm, iid_hbm, hp_hbm, pr_hbm,
              oa_hbm, oi_hbm, os_hbm,
              aid_v, iid_v, hp_v, pr_v, oa_v, oi_v, os_v,
              sem):
    wid = lax.axis_index("s") * NC + lax.axis_index("c")
    base = wid * BPW
    cps = [
        pltpu.async_copy(aid_hbm.at[pl.ds(base, BPW)], aid_v, sem),
        pltpu.async_copy(iid_hbm.at[pl.ds(base, BPW)], iid_v, sem),
        pltpu.async_copy(hp_hbm.at[pl.ds(2 * base, 2 * BPW)], hp_v, sem),
        pltpu.async_copy(pr_hbm, pr_v, sem),
    ]
    for cp in cps:
      cp.wait()

    WB = 464
    w00 = pr_v[pl.ds(WB + 0 * L, L)]
    w01 = pr_v[pl.ds(WB + 1 * L, L)]
    w10 = pr_v[pl.ds(WB + 2 * L, L)]
    w11 = pr_v[pl.ds(WB + 3 * L, L)]
    b0 = pr_v[pl.ds(WB + 4 * L, L)]
    b1 = pr_v[pl.ds(WB + 5 * L, L)]

    @plsc.parallel_loop(0, CHUNKS, unroll=8)
    def body(c):
      off = c * L
      aid = aid_v[pl.ds(off, L)]
      iid = iid_v[pl.ds(off, L)]
      gl = off // 128
      lo = off % 128
      for d in range(D_ANIMAL):
        oa_v[pl.ds(gl * 1024 + d * 128 + lo, L)] = plsc.load_gather(
            pr_v, [aid * D_ANIMAL + d])
      for d in range(D_ITEM):
        oi_v[pl.ds(gl * 512 + d * 128 + lo, L)] = plsc.load_gather(
            pr_v, [400 + iid * D_ITEM + d])
      hp = hp_v[pl.ds(gl * 256 + lo, L)]
      atk = hp_v[pl.ds(gl * 256 + 128 + lo, L)]
      os_v[pl.ds(gl * 256 + lo, L)] = hp * w00 + atk * w01 + b0
      os_v[pl.ds(gl * 256 + 128 + lo, L)] = hp * w10 + atk * w11 + b1

    ocps = [
        pltpu.async_copy(oa_v, oa_hbm.at[pl.ds(8 * base, 8 * BPW)], sem),
        pltpu.async_copy(oi_v, oi_hbm.at[pl.ds(4 * base, 4 * BPW)], sem),
        pltpu.async_copy(os_v, os_hbm.at[pl.ds(2 * base, 2 * BPW)], sem),
    ]
    for cp in ocps:
      cp.wait()

  return sc_unit


def kernel(animal_id, item_id, hp_atk, table_animal, table_item, W, b):
  wb = jnp.broadcast_to(
      jnp.concatenate([W.reshape(-1), b]).reshape(6, 1), (6, L)
  ).reshape(-1)
  params = jnp.concatenate([
      table_animal.reshape(-1),
      table_item.reshape(-1),
      jnp.zeros((4,), jnp.float32),
      wb,
  ])
  oa_f, oi_f, os_f = _build_sc_unit()(
      animal_id,
      item_id,
      hp_atk.T.reshape(2, 128, 128).transpose(1, 0, 2).reshape(-1),
      params,
  )
  oa = oa_f.reshape(128, 8, 128)[:, :D_ANIMAL, :].transpose(1, 0, 2)
  oi = oi_f.reshape(128, 4, 128)[:, :D_ITEM, :].transpose(1, 0, 2)
  os_ = os_f.reshape(128, 2, 128).transpose(1, 0, 2)
  return (
      oa.reshape(D_ANIMAL, B).T,
      oi.reshape(D_ITEM, B).T,
      os_.reshape(2, B).T,
  )


# speedup vs baseline: 1.0649x; 1.0028x over previous
import functools

import jax
import jax.numpy as jnp
from jax import lax
from jax.experimental import pallas as pl
from jax.experimental.pallas import tpu as pltpu
from jax.experimental.pallas import tpu_sc as plsc

B = 16384
N_ANIMAL, D_ANIMAL = 80, 5
N_ITEM, D_ITEM = 20, 3
NC, NS, L = 2, 16, 16
NW = NC * NS
BPW = B // NW
CHUNKS = BPW // L


@functools.cache
def _build_sc_unit():
  mesh = plsc.VectorSubcoreMesh(
      core_axis_name="c", subcore_axis_name="s", num_cores=NC, num_subcores=NS
  )

  @functools.partial(
      pl.kernel,
      out_type=(
          jax.ShapeDtypeStruct((8 * B,), jnp.float32),
          jax.ShapeDtypeStruct((4 * B,), jnp.float32),
          jax.ShapeDtypeStruct((2 * B,), jnp.float32),
      ),
      mesh=mesh,
      scratch_types=[
          pltpu.VMEM((BPW,), jnp.int32),
          pltpu.VMEM((BPW,), jnp.int32),
          pltpu.VMEM((2 * BPW,), jnp.float32),
          pltpu.VMEM((560,), jnp.float32),
          pltpu.VMEM((8 * BPW,), jnp.float32),
          pltpu.VMEM((4 * BPW,), jnp.float32),
          pltpu.VMEM((2 * BPW,), jnp.float32),
          pltpu.SemaphoreType.DMA,
      ],
      compiler_params=pltpu.CompilerParams(
          needs_layout_passes=False,
          use_tc_tiling_on_sc=False,
          disable_bounds_checks=True,
          disable_semaphore_checks=True,
      ),
  )
  def sc_unit(aid_hbm, iid_hbm, hp_hbm, pr_hbm,
              oa_hbm, oi_hbm, os_hbm,
              aid_v, iid_v, hp_v, pr_v, oa_v, oi_v, os_v,
              sem):
    wid = lax.axis_index("s") * NC + lax.axis_index("c")
    base = wid * BPW
    cps = [
        pltpu.async_copy(aid_hbm.at[pl.ds(base, BPW)], aid_v, sem),
        pltpu.async_copy(iid_hbm.at[pl.ds(base, BPW)], iid_v, sem),
        pltpu.async_copy(hp_hbm.at[pl.ds(2 * base, 2 * BPW)], hp_v, sem),
        pltpu.async_copy(pr_hbm, pr_v, sem),
    ]
    for cp in cps:
      cp.wait()

    WB = 464
    w00 = pr_v[pl.ds(WB + 0 * L, L)]
    w01 = pr_v[pl.ds(WB + 1 * L, L)]
    w10 = pr_v[pl.ds(WB + 2 * L, L)]
    w11 = pr_v[pl.ds(WB + 3 * L, L)]
    b0 = pr_v[pl.ds(WB + 4 * L, L)]
    b1 = pr_v[pl.ds(WB + 5 * L, L)]

    @plsc.parallel_loop(0, CHUNKS, unroll=2)
    def body(c):
      off = c * L
      aid = aid_v[pl.ds(off, L)]
      iid = iid_v[pl.ds(off, L)]
      gl = off // 128
      lo = off % 128
      for d in range(D_ANIMAL):
        oa_v[pl.ds(gl * 1024 + d * 128 + lo, L)] = plsc.load_gather(
            pr_v, [aid * D_ANIMAL + d])
      for d in range(D_ITEM):
        oi_v[pl.ds(gl * 512 + d * 128 + lo, L)] = plsc.load_gather(
            pr_v, [400 + iid * D_ITEM + d])
      hp = hp_v[pl.ds(gl * 256 + lo, L)]
      atk = hp_v[pl.ds(gl * 256 + 128 + lo, L)]
      os_v[pl.ds(gl * 256 + lo, L)] = hp * w00 + atk * w01 + b0
      os_v[pl.ds(gl * 256 + 128 + lo, L)] = hp * w10 + atk * w11 + b1

    ocps = [
        pltpu.async_copy(oa_v, oa_hbm.at[pl.ds(8 * base, 8 * BPW)], sem),
        pltpu.async_copy(oi_v, oi_hbm.at[pl.ds(4 * base, 4 * BPW)], sem),
        pltpu.async_copy(os_v, os_hbm.at[pl.ds(2 * base, 2 * BPW)], sem),
    ]
    for cp in ocps:
      cp.wait()

  return sc_unit


def kernel(animal_id, item_id, hp_atk, table_animal, table_item, W, b):
  wb = jnp.broadcast_to(
      jnp.concatenate([W.reshape(-1), b]).reshape(6, 1), (6, L)
  ).reshape(-1)
  params = jnp.concatenate([
      table_animal.reshape(-1),
      table_item.reshape(-1),
      jnp.zeros((4,), jnp.float32),
      wb,
  ])
  oa_f, oi_f, os_f = _build_sc_unit()(
      animal_id,
      item_id,
      hp_atk.T.reshape(2, 128, 128).transpose(1, 0, 2).reshape(-1),
      params,
  )
  oa = oa_f.reshape(128, 8, 128)[:, :D_ANIMAL, :].transpose(1, 0, 2)
  oi = oi_f.reshape(128, 4, 128)[:, :D_ITEM, :].transpose(1, 0, 2)
  os_ = os_f.reshape(128, 2, 128).transpose(1, 0, 2)
  return (
      oa.reshape(D_ANIMAL, B).T,
      oi.reshape(D_ITEM, B).T,
      os_.reshape(2, B).T,
  )


# speedup vs baseline: 1.1104x; 1.0427x over previous
import functools

import jax
import jax.numpy as jnp
from jax import lax
from jax.experimental import pallas as pl
from jax.experimental.pallas import tpu as pltpu
from jax.experimental.pallas import tpu_sc as plsc

B = 16384
N_ANIMAL, D_ANIMAL = 80, 5
N_ITEM, D_ITEM = 20, 3
NC, NS, L = 2, 16, 16
NW = NC * NS
BPW = B // NW
CHUNKS = BPW // L
NG = B // 128


@functools.cache
def _build_sc_unit():
  mesh = plsc.VectorSubcoreMesh(
      core_axis_name="c", subcore_axis_name="s", num_cores=NC, num_subcores=NS
  )

  @functools.partial(
      pl.kernel,
      out_type=(
          jax.ShapeDtypeStruct((8 * B,), jnp.float32),
          jax.ShapeDtypeStruct((4 * B,), jnp.float32),
      ),
      mesh=mesh,
      scratch_types=[
          pltpu.VMEM((BPW,), jnp.int32),
          pltpu.VMEM((BPW,), jnp.int32),
          pltpu.VMEM((464,), jnp.float32),
          pltpu.VMEM((8 * BPW,), jnp.float32),
          pltpu.VMEM((4 * BPW,), jnp.float32),
          pltpu.SemaphoreType.DMA,
      ],
      compiler_params=pltpu.CompilerParams(
          needs_layout_passes=False,
          use_tc_tiling_on_sc=False,
          disable_bounds_checks=True,
          disable_semaphore_checks=True,
      ),
  )
  def sc_unit(aid_hbm, iid_hbm, pr_hbm, oa_hbm, oi_hbm,
              aid_v, iid_v, pr_v, oa_v, oi_v, sem):
    wid = lax.axis_index("s") * NC + lax.axis_index("c")
    base = wid * BPW
    cps = [
        pltpu.async_copy(aid_hbm.at[pl.ds(base, BPW)], aid_v, sem),
        pltpu.async_copy(iid_hbm.at[pl.ds(base, BPW)], iid_v, sem),
        pltpu.async_copy(pr_hbm, pr_v, sem),
    ]
    for cp in cps:
      cp.wait()

    @plsc.parallel_loop(0, CHUNKS, unroll=2)
    def body(c):
      off = c * L
      aid = aid_v[pl.ds(off, L)]
      iid = iid_v[pl.ds(off, L)]
      gl = off // 128
      lo = off % 128
      for d in range(D_ANIMAL):
        oa_v[pl.ds(gl * 1024 + d * 128 + lo, L)] = plsc.load_gather(
            pr_v, [aid * D_ANIMAL + d])
      for d in range(D_ITEM):
        oi_v[pl.ds(gl * 512 + d * 128 + lo, L)] = plsc.load_gather(
            pr_v, [400 + iid * D_ITEM + d])

    ocps = [
        pltpu.async_copy(oa_v, oa_hbm.at[pl.ds(8 * base, 8 * BPW)], sem),
        pltpu.async_copy(oi_v, oi_hbm.at[pl.ds(4 * base, 4 * BPW)], sem),
    ]
    for cp in ocps:
      cp.wait()

  return sc_unit


def _tc_stats(x_ref, w_ref, b_ref, o_ref):
  x = x_ref[...]
  up = jnp.concatenate([x[1:], x[:1]], axis=0)
  dn = jnp.concatenate([x[-1:], x[:-1]], axis=0)
  w00 = w_ref[0, 0]
  w01 = w_ref[0, 1]
  w10 = w_ref[1, 0]
  w11 = w_ref[1, 1]
  b0 = b_ref[0]
  b1 = b_ref[1]
  even = (lax.broadcasted_iota(jnp.int32, (256, 128), 0) % 2) == 0
  o_ref[...] = jnp.where(
      even, x * w00 + up * w01 + b0, dn * w10 + x * w11 + b1)


@functools.cache
def _build_tc_stats():
  return pl.pallas_call(
      _tc_stats,
      out_shape=jax.ShapeDtypeStruct((256, 128), jnp.float32),
      in_specs=[
          pl.BlockSpec(memory_space=pltpu.VMEM),
          pl.BlockSpec(memory_space=pltpu.SMEM),
          pl.BlockSpec(memory_space=pltpu.SMEM),
      ],
      out_specs=pl.BlockSpec(memory_space=pltpu.VMEM),
  )


def kernel(animal_id, item_id, hp_atk, table_animal, table_item, W, b):
  params = jnp.concatenate([
      table_animal.reshape(-1),
      table_item.reshape(-1),
      jnp.zeros((4,), jnp.float32),
  ])
  oa_f, oi_f = _build_sc_unit()(animal_id, item_id, params)

  x = hp_atk.T.reshape(2, NG, 128).transpose(1, 0, 2).reshape(256, 128)
  st = _build_tc_stats()(x, W, b)
  os_ = st.reshape(NG, 2, 128).transpose(1, 0, 2).reshape(2, B)

  oa = oa_f.reshape(NG, 8, 128)[:, :D_ANIMAL, :].transpose(1, 0, 2)
  oi = oi_f.reshape(NG, 4, 128)[:, :D_ITEM, :].transpose(1, 0, 2)
  return (
      oa.reshape(D_ANIMAL, B).T,
      oi.reshape(D_ITEM, B).T,
      os_.T,
  )
